# baseline (speedup 1.0000x reference)
import functools

import jax
import jax.numpy as jnp
from jax.experimental import pallas as pl
from jax.experimental.pallas import tpu as pltpu

_M_BLK = 1024


def _mlp_body(emb_ref, small_ref, mask_ref, w1a_ref, w1b_ref, b1_ref,
              w2_ref, b2_ref, out_ref):
    xe = emb_ref[...]
    xs = small_ref[...]
    dn = (((1,), (0,)), ((), ()))
    h = jax.lax.dot_general(xe, w1a_ref[...], dn,
                            preferred_element_type=jnp.float32)
    h = h + jax.lax.dot_general(xs, w1b_ref[...], dn,
                                preferred_element_type=jnp.float32)
    h = jnp.maximum(h + b1_ref[...], 0.0).astype(jnp.bfloat16)
    out = jax.lax.dot_general(h, w2_ref[...], dn,
                              preferred_element_type=jnp.float32)
    out_ref[...] = (out + b2_ref[...]) * mask_ref[...]


@functools.partial(jax.jit, static_argnames=("interpret",))
def kernel(embeddings, visibility_scores, bbox_ltwh, keypoints_xyc,
           feats_masks, W1, b1, W2, b2, interpret=False):
    B, N, E = embeddings.shape
    M = B * N
    F = W1.shape[1]
    T = W2.shape[1]

    kp_flat = keypoints_xyc.reshape(B, N, -1)
    small = jnp.concatenate([visibility_scores, bbox_ltwh, kp_flat],
                            axis=-1).reshape(M, -1)
    S = small.shape[-1]

    emb2 = embeddings.reshape(M, E).astype(jnp.bfloat16)
    small = small.astype(jnp.bfloat16)
    mask_f = feats_masks.reshape(M, 1).astype(jnp.float32)
    w1a = W1[:E].astype(jnp.bfloat16)
    w1b = W1[E:].astype(jnp.bfloat16)
    w2 = W2.astype(jnp.bfloat16)
    b1r = b1.reshape(1, F)
    b2r = b2.reshape(1, T)

    grid = (M // _M_BLK,)
    out = pl.pallas_call(
        _mlp_body,
        grid=grid,
        in_specs=[
            pl.BlockSpec((_M_BLK, E), lambda i: (i, 0)),
            pl.BlockSpec((_M_BLK, S), lambda i: (i, 0)),
            pl.BlockSpec((_M_BLK, 1), lambda i: (i, 0)),
            pl.BlockSpec((E, F), lambda i: (0, 0)),
            pl.BlockSpec((S, F), lambda i: (0, 0)),
            pl.BlockSpec((1, F), lambda i: (0, 0)),
            pl.BlockSpec((F, T), lambda i: (0, 0)),
            pl.BlockSpec((1, T), lambda i: (0, 0)),
        ],
        out_specs=pl.BlockSpec((_M_BLK, T), lambda i: (i, 0)),
        out_shape=jax.ShapeDtypeStruct((M, T), jnp.float32),
        compiler_params=pltpu.CompilerParams(
            dimension_semantics=("arbitrary",),
        ),
        interpret=interpret,
    )(emb2, small, mask_f, w1a, w1b, b1r, w2, b2r)
    return out.reshape(B, N, T)


# speedup vs baseline: 1.0104x; 1.0104x over previous
import functools

import jax
import jax.numpy as jnp
from jax.experimental import pallas as pl
from jax.experimental.pallas import tpu as pltpu

_M_BLK = 1024


def _mlp_body(emb_ref, small_ref, mask_ref, w1a_ref, w1b_ref, b1_ref,
              w2_ref, b2_ref, out_ref):
    xe = emb_ref[...].astype(jnp.bfloat16)
    xs = small_ref[...].astype(jnp.bfloat16)
    dn = (((1,), (0,)), ((), ()))
    h = jax.lax.dot_general(xe, w1a_ref[...], dn,
                            preferred_element_type=jnp.float32)
    h = h + jax.lax.dot_general(xs, w1b_ref[...], dn,
                                preferred_element_type=jnp.float32)
    h = jnp.maximum(h + b1_ref[...], 0.0).astype(jnp.bfloat16)
    out = jax.lax.dot_general(h, w2_ref[...], dn,
                              preferred_element_type=jnp.float32)
    out_ref[...] = (out + b2_ref[...]) * mask_ref[...]


@functools.partial(jax.jit, static_argnames=("interpret",))
def kernel(embeddings, visibility_scores, bbox_ltwh, keypoints_xyc,
           feats_masks, W1, b1, W2, b2, interpret=False):
    B, N, E = embeddings.shape
    M = B * N
    F = W1.shape[1]
    T = W2.shape[1]

    kp_flat = keypoints_xyc.reshape(B, N, -1)
    small = jnp.concatenate([visibility_scores, bbox_ltwh, kp_flat],
                            axis=-1).reshape(M, -1)
    S = small.shape[-1]

    emb2 = embeddings.reshape(M, E)
    mask_f = feats_masks.reshape(M, 1).astype(jnp.float32)
    w1a = W1[:E].astype(jnp.bfloat16)
    w1b = W1[E:].astype(jnp.bfloat16)
    w2 = W2.astype(jnp.bfloat16)
    b1r = b1.reshape(1, F)
    b2r = b2.reshape(1, T)

    grid = (M // _M_BLK,)
    out = pl.pallas_call(
        _mlp_body,
        grid=grid,
        in_specs=[
            pl.BlockSpec((_M_BLK, E), lambda i: (i, 0)),
            pl.BlockSpec((_M_BLK, S), lambda i: (i, 0)),
            pl.BlockSpec((_M_BLK, 1), lambda i: (i, 0)),
            pl.BlockSpec((E, F), lambda i: (0, 0)),
            pl.BlockSpec((S, F), lambda i: (0, 0)),
            pl.BlockSpec((1, F), lambda i: (0, 0)),
            pl.BlockSpec((F, T), lambda i: (0, 0)),
            pl.BlockSpec((1, T), lambda i: (0, 0)),
        ],
        out_specs=pl.BlockSpec((_M_BLK, T), lambda i: (i, 0)),
        out_shape=jax.ShapeDtypeStruct((M, T), jnp.float32),
        compiler_params=pltpu.CompilerParams(
            dimension_semantics=("arbitrary",),
        ),
        interpret=interpret,
    )(emb2, small, mask_f, w1a, w1b, b1r, w2, b2r)
    return out.reshape(B, N, T)
